# baseline (speedup 1.0000x reference)
import jax
import jax.numpy as jnp
from jax.experimental import pallas as pl
from jax.experimental.pallas import tpu as pltpu

X, Y, P = 34, 34, 2
B, T = 128, 32
N_IN = X * Y * P
H = 512
OUT = 10
M_MAX = 16
MASK = 15
ALPHA = 0.001
TSUB = 4
NBLK = T // TSUB


def _snn_block(frames_ref, w01_ref, w23_ref, packa0_ref, wcomb_ref,
               tau1_ref, vth1_ref, mm0_ref, mm1_ref, mm1t_ref,
               out_float_ref, spikes_ref, dmap0_ref, dmap1_ref,
               p0_ref, p1_ref, mf1_ref, tl_ref, cnt0_ref, cnt1_ref,
               of_acc_ref, sc_acc_ref):
    blk = pl.program_id(0)

    @pl.when(blk == 0)
    def _init():
        p0_ref[...] = jnp.zeros_like(p0_ref)
        p1_ref[...] = jnp.zeros_like(p1_ref)
        mf1_ref[...] = jnp.zeros_like(mf1_ref)
        tl_ref[...] = jnp.zeros_like(tl_ref)
        cnt0_ref[...] = jnp.zeros_like(cnt0_ref)
        cnt1_ref[...] = jnp.zeros_like(cnt1_ref)
        of_acc_ref[...] = jnp.zeros_like(of_acc_ref)
        sc_acc_ref[...] = jnp.zeros_like(sc_acc_ref)

    iota16 = jax.lax.broadcasted_iota(jnp.int32, (1, M_MAX), 1)

    for j in range(TSUB):
        t = blk * TSUB + j

        x = frames_ref[:, j, :, :, :]
        s01 = jnp.sum(x * w01_ref[...], axis=(1, 2, 3)).reshape(B, 1)
        s23 = jnp.sum(x * w23_ref[...], axis=(1, 2, 3)).reshape(B, 1)
        cntb = jnp.sum(x, axis=(1, 2, 3)).reshape(B, 1)
        acc0 = ((s01 & 1)
                | (((s01 >> 16) & 1) << 1)
                | ((s23 & 1) << 2)
                | (((s23 >> 16) & 1) << 3))
        total = jnp.sum(cntb, axis=0, keepdims=True)
        g = total > 0

        tl = tl_ref[...]
        dt = (t - tl) & MASK
        tl_ref[...] = jnp.where(g, jnp.full_like(tl, t), tl)

        p0 = p0_ref[...]
        p0 = jnp.where(g, (p0 ^ dt ^ (acc0 & MASK)) & MASK, p0)
        p0_ref[...] = p0

        a1i = (packa0_ref[...] >> p0) & 1
        a1f = a1i.astype(jnp.float32)

        oh0 = jnp.sum((p0 == iota16).astype(jnp.int32), axis=0, keepdims=True)
        cnt0_ref[...] = cnt0_ref[...] + jnp.where(g, oh0, 0)

        rc = jax.lax.dot_general(a1f, wcomb_ref[...], (((1,), (0,)), ((), ())),
                                 preferred_element_type=jnp.float32)
        acc1 = jnp.zeros((B, 1), jnp.int32)
        for k in range(4):
            sk = rc[:, k:k + 1].astype(jnp.int32)
            acc1 = acc1 | ((sk & 1) << k)
        syn = rc[:, 4:4 + OUT]

        dtf = dt.astype(jnp.float32)
        decay = jnp.exp(-tau1_ref[...] * dtf)
        mf1 = mf1_ref[...]
        mf1_ref[...] = jnp.where(g, mf1 * decay + syn, mf1)

        p1 = p1_ref[...]
        p1 = jnp.where(g, (p1 ^ dt ^ (acc1 & MASK)) & MASK, p1)
        p1_ref[...] = p1

        v2 = jnp.zeros((B, OUT), jnp.float32)
        for m in range(M_MAX):
            v2 = jnp.where(p1 == m, mm1t_ref[m:m + 1, :], v2)
        a2 = v2 >= vth1_ref[...]

        oh1 = jnp.sum((p1 == iota16).astype(jnp.int32), axis=0, keepdims=True)
        cnt1_ref[...] = cnt1_ref[...] + jnp.where(g, oh1, 0)

        sc_acc_ref[...] = sc_acc_ref[...] + jnp.where(g, a2.astype(jnp.int32),
                                                      0)
        of_acc_ref[...] = of_acc_ref[...] + jnp.where(g, v2, 0.0)

    @pl.when(blk == NBLK - 1)
    def _fin():
        c0 = cnt0_ref[...].astype(jnp.float32)
        c1 = cnt1_ref[...].astype(jnp.float32)
        mv0 = jax.lax.broadcasted_iota(
            jnp.int32, (H, M_MAX), 1).astype(jnp.float32)
        mv1 = jax.lax.broadcasted_iota(
            jnp.int32, (OUT, M_MAX), 1).astype(jnp.float32)
        dmap0_ref[...] = ALPHA * c0 * (mv0 - mm0_ref[...])
        dmap1 = ALPHA * c1 * (mv1 - mm1_ref[...])
        dmap1_ref[...] = dmap1
        mvt = jax.lax.broadcasted_iota(
            jnp.int32, (M_MAX, OUT), 0).astype(jnp.float32)
        c1t = cnt1_ref[...].astype(jnp.float32).reshape(M_MAX, 1)
        dmap1t = ALPHA * c1t * (mvt - mm1t_ref[...])
        dsum = jnp.sum(dmap1t, axis=0, keepdims=True)
        out_float_ref[...] = of_acc_ref[...] + mf1_ref[...] + dsum
        spikes_ref[...] = sc_acc_ref[...]


def kernel(frames_batch, tau0, vth0, nid0, memmap0, synw0,
           tau1, vth1, nid1, memmap1, synw1):
    del tau0, nid1, synw0

    ii = jnp.arange(N_IN, dtype=jnp.int32).reshape(P, X, Y)
    w01 = (ii & 1) | (((ii >> 1) & 1) << 16)
    w23 = ((ii >> 2) & 1) | (((ii >> 3) & 1) << 16)

    packa0 = jnp.sum(
        (memmap0 >= vth0[:, None]).astype(jnp.int32)
        << jnp.arange(M_MAX, dtype=jnp.int32)[None, :], axis=1)
    packa0_2d = packa0.reshape(1, H)

    nb = jnp.stack([(nid0 >> k) & 1 for k in range(4)], axis=1)
    wcomb = jnp.concatenate([nb.astype(jnp.float32), synw1], axis=1)

    mm1t = memmap1.T
    vth1_2d = vth1.reshape(1, OUT)
    tau1_2d = tau1.reshape(1, OUT)

    full = lambda shape: pl.BlockSpec(shape, lambda i: tuple(0 for _ in shape))
    out_shapes = (
        jax.ShapeDtypeStruct((B, OUT), jnp.float32),
        jax.ShapeDtypeStruct((B, OUT), jnp.int32),
        jax.ShapeDtypeStruct((H, M_MAX), jnp.float32),
        jax.ShapeDtypeStruct((OUT, M_MAX), jnp.float32),
    )
    out_float, spikes, dmap0, dmap1 = pl.pallas_call(
        _snn_block,
        grid=(NBLK,),
        in_specs=[
            pl.BlockSpec((B, TSUB, P, X, Y), lambda i: (0, i, 0, 0, 0)),
            pl.BlockSpec((P, X, Y), lambda i: (0, 0, 0)),
            pl.BlockSpec((P, X, Y), lambda i: (0, 0, 0)),
            full((1, H)),
            full((H, 4 + OUT)),
            full((1, OUT)),
            full((1, OUT)),
            full((H, M_MAX)),
            full((OUT, M_MAX)),
            full((M_MAX, OUT)),
        ],
        out_specs=(
            full((B, OUT)),
            full((B, OUT)),
            full((H, M_MAX)),
            full((OUT, M_MAX)),
        ),
        out_shape=out_shapes,
        scratch_shapes=[
            pltpu.VMEM((B, 1), jnp.int32),
            pltpu.VMEM((B, 1), jnp.int32),
            pltpu.VMEM((B, OUT), jnp.float32),
            pltpu.VMEM((1, 1), jnp.int32),
            pltpu.VMEM((1, M_MAX), jnp.int32),
            pltpu.VMEM((1, M_MAX), jnp.int32),
            pltpu.VMEM((B, OUT), jnp.float32),
            pltpu.VMEM((B, OUT), jnp.int32),
        ],
    )(frames_batch, w01, w23, packa0_2d, wcomb,
      tau1_2d, vth1_2d, memmap0, memmap1, mm1t)
    return out_float, spikes, dmap0, dmap1


# speedup vs baseline: 1.4361x; 1.4361x over previous
import jax
import jax.numpy as jnp
from jax.experimental import pallas as pl
from jax.experimental.pallas import tpu as pltpu

X, Y, P = 34, 34, 2
B, T = 128, 32
N_IN = X * Y * P
H = 512
OUT = 10
M_MAX = 16
MASK = 15
ALPHA = 0.001
N_CHUNKS = 2
S = T // N_CHUNKS


def _make_step(t_base):
    def _snn_step(frames_ref, bm_ref, packa0_ref, wcomb_ref,
                  tau1_ref, vth1_ref, mm0_ref, mm1_ref, mm1t_ref,
                  p0i_ref, p1i_ref, mf1i_ref, tli_ref, cnt0i_ref, cnt1i_ref,
                  ofi_ref, sci_ref,
                  p0o_ref, p1o_ref, mf1o_ref, tlo_ref, cnt0o_ref, cnt1o_ref,
                  ofo_ref, sco_ref,
                  out_float_ref, spikes_ref, dmap0_ref, dmap1_ref,
                  p0_ref, p1_ref, mf1_ref, tl_ref, cnt0_ref, cnt1_ref,
                  of_acc_ref, sc_acc_ref):
        tc = pl.program_id(0)
        t = tc + t_base

        @pl.when(tc == 0)
        def _init():
            p0_ref[...] = p0i_ref[...]
            p1_ref[...] = p1i_ref[...]
            mf1_ref[...] = mf1i_ref[...]
            tl_ref[...] = tli_ref[...]
            cnt0_ref[...] = cnt0i_ref[...]
            cnt1_ref[...] = cnt1i_ref[...]
            of_acc_ref[...] = ofi_ref[...]
            sc_acc_ref[...] = sci_ref[...]

        x = frames_ref[:, 0, 0, :]
        xb = x.astype(jnp.float32)
        r = jax.lax.dot_general(xb, bm_ref[...], (((1,), (0,)), ((), ())),
                                preferred_element_type=jnp.float32)
        acc0 = jnp.zeros((B, 1), jnp.int32)
        for k in range(4):
            sk = r[:, k:k + 1].astype(jnp.int32)
            acc0 = acc0 | ((sk & 1) << k)
        total = jnp.sum(r[:, 4:5], axis=0, keepdims=True)
        g = total > 0.0

        tl = tl_ref[...]
        dt = (t - tl) & MASK
        tl_ref[...] = jnp.where(g, jnp.full_like(tl, t), tl)

        p0 = p0_ref[...]
        p0 = jnp.where(g, (p0 ^ dt ^ (acc0 & MASK)) & MASK, p0)
        p0_ref[...] = p0

        a1i = (packa0_ref[...] >> p0) & 1
        a1f = a1i.astype(jnp.float32)

        iota16 = jax.lax.broadcasted_iota(jnp.int32, (1, M_MAX), 1)
        oh0 = jnp.sum((p0 == iota16).astype(jnp.int32), axis=0, keepdims=True)
        cnt0_ref[...] = cnt0_ref[...] + jnp.where(g, oh0, 0)

        rc = jax.lax.dot_general(a1f, wcomb_ref[...],
                                 (((1,), (0,)), ((), ())),
                                 preferred_element_type=jnp.float32)
        acc1 = jnp.zeros((B, 1), jnp.int32)
        for k in range(4):
            sk = rc[:, k:k + 1].astype(jnp.int32)
            acc1 = acc1 | ((sk & 1) << k)
        syn = rc[:, 4:4 + OUT]

        dtf = dt.astype(jnp.float32)
        decay = jnp.exp(-tau1_ref[...] * dtf)
        mf1 = mf1_ref[...]
        mf1_ref[...] = jnp.where(g, mf1 * decay + syn, mf1)

        p1 = p1_ref[...]
        p1 = jnp.where(g, (p1 ^ dt ^ (acc1 & MASK)) & MASK, p1)
        p1_ref[...] = p1

        v2 = jnp.zeros((B, OUT), jnp.float32)
        for m in range(M_MAX):
            v2 = jnp.where(p1 == m, mm1t_ref[m:m + 1, :], v2)
        a2 = v2 >= vth1_ref[...]

        oh1 = jnp.sum((p1 == iota16).astype(jnp.int32), axis=0, keepdims=True)
        cnt1_ref[...] = cnt1_ref[...] + jnp.where(g, oh1, 0)

        sc_acc_ref[...] = sc_acc_ref[...] + jnp.where(g, a2.astype(jnp.int32),
                                                      0)
        of_acc_ref[...] = of_acc_ref[...] + jnp.where(g, v2, 0.0)

        @pl.when(tc == S - 1)
        def _fin():
            p0o_ref[...] = p0_ref[...]
            p1o_ref[...] = p1_ref[...]
            mf1o_ref[...] = mf1_ref[...]
            tlo_ref[...] = tl_ref[...]
            cnt0o_ref[...] = cnt0_ref[...]
            cnt1o_ref[...] = cnt1_ref[...]
            ofo_ref[...] = of_acc_ref[...]
            sco_ref[...] = sc_acc_ref[...]
            c0 = cnt0_ref[...].astype(jnp.float32)
            c1 = cnt1_ref[...].astype(jnp.float32)
            mv0 = jax.lax.broadcasted_iota(
                jnp.int32, (H, M_MAX), 1).astype(jnp.float32)
            mv1 = jax.lax.broadcasted_iota(
                jnp.int32, (OUT, M_MAX), 1).astype(jnp.float32)
            dmap0_ref[...] = ALPHA * c0 * (mv0 - mm0_ref[...])
            dmap1 = ALPHA * c1 * (mv1 - mm1_ref[...])
            dmap1_ref[...] = dmap1
            mvt = jax.lax.broadcasted_iota(
                jnp.int32, (M_MAX, OUT), 0).astype(jnp.float32)
            c1t = cnt1_ref[...].astype(jnp.float32).reshape(M_MAX, 1)
            dmap1t = ALPHA * c1t * (mvt - mm1t_ref[...])
            dsum = jnp.sum(dmap1t, axis=0, keepdims=True)
            out_float_ref[...] = of_acc_ref[...] + mf1_ref[...] + dsum
            spikes_ref[...] = sc_acc_ref[...]

    return _snn_step


def kernel(frames_batch, tau0, vth0, nid0, memmap0, synw0,
           tau1, vth1, nid1, memmap1, synw1):
    del tau0, nid1, synw0

    ii = jnp.arange(N_IN, dtype=jnp.int32)
    bm = jnp.stack([(ii >> k) & 1 for k in range(4)]
                   + [jnp.ones_like(ii)] + [jnp.zeros_like(ii)] * 3,
                   axis=1).astype(jnp.float32)

    packa0 = jnp.sum(
        (memmap0 >= vth0[:, None]).astype(jnp.int32)
        << jnp.arange(M_MAX, dtype=jnp.int32)[None, :], axis=1)
    packa0_2d = packa0.reshape(1, H)

    nb = jnp.stack([(nid0 >> k) & 1 for k in range(4)], axis=1)
    wcomb = jnp.concatenate([nb.astype(jnp.float32), synw1], axis=1)

    mm1t = memmap1.T
    vth1_2d = vth1.reshape(1, OUT)
    tau1_2d = tau1.reshape(1, OUT)

    full = lambda shape: pl.BlockSpec(shape, lambda t: tuple(0 for _ in shape))
    state_shapes = (
        jax.ShapeDtypeStruct((B, 1), jnp.int32),
        jax.ShapeDtypeStruct((B, 1), jnp.int32),
        jax.ShapeDtypeStruct((B, OUT), jnp.float32),
        jax.ShapeDtypeStruct((1, 1), jnp.int32),
        jax.ShapeDtypeStruct((1, M_MAX), jnp.int32),
        jax.ShapeDtypeStruct((1, M_MAX), jnp.int32),
        jax.ShapeDtypeStruct((B, OUT), jnp.float32),
        jax.ShapeDtypeStruct((B, OUT), jnp.int32),
    )
    final_shapes = (
        jax.ShapeDtypeStruct((B, OUT), jnp.float32),
        jax.ShapeDtypeStruct((B, OUT), jnp.int32),
        jax.ShapeDtypeStruct((H, M_MAX), jnp.float32),
        jax.ShapeDtypeStruct((OUT, M_MAX), jnp.float32),
    )
    state_specs = tuple(full(s.shape) for s in state_shapes)
    const_specs = [
        full((N_IN, 8)),
        full((1, H)),
        full((H, 4 + OUT)),
        full((1, OUT)),
        full((1, OUT)),
        full((H, M_MAX)),
        full((OUT, M_MAX)),
        full((M_MAX, OUT)),
    ]
    scratch = [
        pltpu.VMEM((B, 1), jnp.int32),
        pltpu.VMEM((B, 1), jnp.int32),
        pltpu.VMEM((B, OUT), jnp.float32),
        pltpu.VMEM((1, 1), jnp.int32),
        pltpu.VMEM((1, M_MAX), jnp.int32),
        pltpu.VMEM((1, M_MAX), jnp.int32),
        pltpu.VMEM((B, OUT), jnp.float32),
        pltpu.VMEM((B, OUT), jnp.int32),
    ]

    state = tuple(jnp.zeros(s.shape, s.dtype) for s in state_shapes)
    finals = None
    for c in range(N_CHUNKS):
        fr = frames_batch[:, c * S:(c + 1) * S].reshape(B, S, 1, N_IN)
        outs = pl.pallas_call(
            _make_step(c * S),
            grid=(S,),
            in_specs=[pl.BlockSpec((B, 1, 1, N_IN), lambda t: (0, t, 0, 0))]
            + const_specs + list(state_specs),
            out_specs=state_specs + tuple(full(s.shape) for s in final_shapes),
            out_shape=state_shapes + final_shapes,
            scratch_shapes=scratch,
        )(fr, bm, packa0_2d, wcomb, tau1_2d, vth1_2d, memmap0, memmap1, mm1t,
          *state)
        state = outs[:8]
        finals = outs[8:]
    return finals
